# baseline (speedup 1.0000x reference)
import functools
import math

import numpy as np
import jax
import jax.numpy as jnp
from jax import lax
from jax.experimental import pallas as pl
from jax.experimental.pallas import tpu as pltpu
from jax.experimental.pallas import tpu_sc as plsc

F32 = jnp.float32
I32 = jnp.int32
NW = 32
EPAD_EXTRA = 512
_EPS = 1e-05


def _sc_mesh():
    return plsc.VectorSubcoreMesh(core_axis_name="c", subcore_axis_name="s")


def _wid():
    return lax.axis_index("s") * 2 + lax.axis_index("c")


@functools.cache
def _build_gather(n_rows_out, d, chunk, two, nbuf):
    n_chunks = n_rows_out // chunk
    assert n_rows_out % chunk == 0 and d % 16 == 0 and chunk % 8 == 0
    n_it = (n_chunks + NW - 1) // NW

    out_t = jax.ShapeDtypeStruct((n_rows_out, d), F32)
    scratch = ([pltpu.VMEM((chunk,), I32) for _ in range(nbuf)]
               + [pltpu.VMEM((chunk, d), F32) for _ in range(nbuf)]
               + [pltpu.SemaphoreType.DMA for _ in range(nbuf)])
    if two:
        scratch += ([pltpu.VMEM((chunk,), I32) for _ in range(nbuf)]
                    + [pltpu.VMEM((chunk, d), F32) for _ in range(nbuf)])

    @functools.partial(
        pl.kernel, mesh=_sc_mesh(),
        out_type=[out_t, out_t] if two else out_t,
        scratch_types=scratch,
    )
    def gk(*refs):
        if two:
            (ta, tb, ia_hbm, ib_hbm, outa, outb) = refs[:6]
            rest = refs[6:]
        else:
            ta, ia_hbm, outa = refs[:3]
            rest = refs[3:]
        ias = rest[:nbuf]
        ras = rest[nbuf:2 * nbuf]
        sems = rest[2 * nbuf:3 * nbuf]
        if two:
            ibs = rest[3 * nbuf:4 * nbuf]
            rbs = rest[4 * nbuf:5 * nbuf]
        wid = _wid()

        def start(it, b):
            cid = it * NW + wid

            @pl.when(cid < n_chunks)
            def _():
                base = cid * chunk
                pltpu.sync_copy(ia_hbm.at[pl.ds(base, chunk)], ias[b])
                pltpu.async_copy(ta.at[ias[b]], ras[b], sems[b])
                if two:
                    pltpu.sync_copy(ib_hbm.at[pl.ds(base, chunk)], ibs[b])
                    pltpu.async_copy(tb.at[ibs[b]], rbs[b], sems[b])

        def finish(it, b):
            cid = it * NW + wid

            @pl.when(cid < n_chunks)
            def _():
                pltpu.make_async_copy(ta.at[ias[b]], ras[b], sems[b]).wait()
                pltpu.sync_copy(ras[b], outa.at[pl.ds(cid * chunk, chunk)])
                if two:
                    pltpu.make_async_copy(tb.at[ibs[b]], rbs[b],
                                          sems[b]).wait()
                    pltpu.sync_copy(rbs[b],
                                    outb.at[pl.ds(cid * chunk, chunk)])

        for b in range(nbuf):
            start(b, b)
        n_grp = max((n_it - nbuf) // nbuf, 0)

        def grp(g, c):
            for b in range(nbuf):
                it = nbuf * g + b
                finish(it, b)
                start(it + nbuf, b)
            return c

        lax.fori_loop(0, n_grp, grp, 0)
        for it in range(nbuf * n_grp, n_it):
            finish(it, it % nbuf)
            start(it + nbuf, it % nbuf)

    return gk


def _sc_gather(table, idx, chunk=256, nbuf=3):
    table = jnp.asarray(table, F32)
    return _build_gather(idx.shape[0], table.shape[1], chunk, False, nbuf)(
        table, idx)


def _sc_gather2(ta, tb, ia, ib, chunk=128, nbuf=3):
    return _build_gather(ia.shape[0], ta.shape[1], chunk, True, nbuf)(
        ta, tb, ia, ib)


@functools.cache
def _build_segreduce(e_pad, width, npw, chunk, op):
    assert width % 16 == 0 and chunk % 8 == 0
    nvec = width // 16
    ident = {"sum": 0.0, "min": float("inf"), "max": float("-inf")}[op]
    red = {"sum": lambda a, b: a + b,
           "min": jnp.minimum, "max": jnp.maximum}[op]

    @functools.partial(
        pl.kernel, mesh=_sc_mesh(),
        out_type=jax.ShapeDtypeStruct((NW * npw * width,), F32),
        scratch_types=[
            pltpu.VMEM((56,), I32),
            pltpu.VMEM((chunk + 16,), I32),
            pltpu.VMEM((npw * width,), F32),
            pltpu.VMEM((chunk * width + 16 * nvec,), F32),
        ],
    )
    def srk(msgs_hbm, seg_hbm, bounds_hbm, out_hbm, bounds_s, seg_s,
            acc_v, chunk_v):
        wid = _wid()
        pltpu.sync_copy(bounds_hbm, bounds_s.at[pl.ds(0, 40)])

        def initb(i, c):
            acc_v[pl.ds(i * 16, 16)] = jnp.full((16,), ident, F32)
            return c

        lax.fori_loop(0, npw * width // 16, initb, 0)
        ident_base = chunk * width
        for jj in range(nvec):
            chunk_v[pl.ds(ident_base + jj * 16, 16)] = jnp.full(
                (16,), ident, F32)

        start = bounds_s[pl.ds(wid, 16)][0]
        end = bounds_s[pl.ds(wid + 1, 16)][0]
        node_base = bounds_s[pl.ds(33, 16)][0]
        abase = (start // 8) * 8
        nch = (end - abase + chunk - 1) // chunk
        base0 = node_base + wid * npw

        def chunk_body(ic, c):
            base = abase + ic * chunk
            pltpu.sync_copy(msgs_hbm.at[pl.ds(base * width, chunk * width)],
                            chunk_v.at[pl.ds(0, chunk * width)])
            pltpu.sync_copy(seg_hbm.at[pl.ds(base, chunk)],
                            seg_s.at[pl.ds(0, chunk)])

            def grp_body(gi, c2):
                segv = seg_s[pl.ds(gi * 16, 16)] - base0
                for l in range(16):
                    j = gi * 16 + l
                    e = base + j
                    node = jnp.clip(segv[l], 0, npw - 1)
                    valid = jnp.logical_and(e >= start, e < end)
                    off = node * width
                    moff = jnp.where(valid, j * width, ident_base)
                    for jj in range(nvec):
                        a = acc_v[pl.ds(off + jj * 16, 16)]
                        m = chunk_v[pl.ds(moff + jj * 16, 16)]
                        acc_v[pl.ds(off + jj * 16, 16)] = red(a, m)
                return c2

            lax.fori_loop(0, chunk // 16, grp_body, 0)
            return c

        lax.fori_loop(0, nch, chunk_body, 0)
        pltpu.sync_copy(acc_v, out_hbm.at[pl.ds(wid * npw * width, npw * width)])

    return srk


def _sc_segreduce(msgs2d, seg_pad, bounds, n_out, width, npw, chunk, op):
    e_pad = seg_pad.shape[0]
    flat = msgs2d.reshape(-1)
    out = _build_segreduce(e_pad, width, npw, chunk, op)(flat, seg_pad, bounds)
    return out.reshape(NW * npw, width)[:n_out]


@functools.cache
def _build_pools(r_pad, n_real, width, nseg, chunk, ops):
    n_chunks = r_pad // chunk
    assert r_pad % chunk == 0 and width % 16 == 0
    n_it = (n_chunks + NW - 1) // NW
    nvec = width // 16
    idents = {"sum": 0.0, "sumsq": 0.0, "min": float("inf"),
              "max": float("-inf")}

    @functools.partial(
        pl.kernel, mesh=_sc_mesh(),
        out_type=[jax.ShapeDtypeStruct((NW * nseg * width,), F32)
                  for _ in ops],
        scratch_types=[
            pltpu.VMEM((chunk + 16,), I32),
            pltpu.VMEM((chunk * width,), F32),
        ] + [pltpu.VMEM((nseg * width,), F32) for _ in ops],
    )
    def pk(table_hbm, seg_hbm, *rest):
        outs = rest[:len(ops)]
        seg_s = rest[len(ops)]
        chunk_v = rest[len(ops) + 1]
        accs = rest[len(ops) + 2:]
        wid = _wid()

        def initb(i, c):
            for k, o in enumerate(ops):
                accs[k][pl.ds(i * 16, 16)] = jnp.full((16,), idents[o], F32)
            return c

        lax.fori_loop(0, nseg * width // 16, initb, 0)

        def body(it, c):
            cid = it * NW + wid

            @pl.when(cid < n_chunks)
            def _():
                base = cid * chunk
                pltpu.sync_copy(
                    table_hbm.at[pl.ds(base * width, chunk * width)], chunk_v)
                pltpu.sync_copy(seg_hbm.at[pl.ds(base, chunk)],
                                seg_s.at[pl.ds(0, chunk)])

                def grp_body(gi, c2):
                    segv = seg_s[pl.ds(gi * 16, 16)]
                    for l in range(16):
                        j = gi * 16 + l

                        @pl.when(base + j < n_real)
                        def _(l=l, j=j):
                            off = segv[l] * width
                            moff = j * width
                            for jj in range(nvec):
                                v = chunk_v[pl.ds(moff + jj * 16, 16)]
                                for k, o in enumerate(ops):
                                    a = accs[k][pl.ds(off + jj * 16, 16)]
                                    if o == "sum":
                                        a = a + v
                                    elif o == "sumsq":
                                        a = a + v * v
                                    elif o == "max":
                                        a = jnp.maximum(a, v)
                                    else:
                                        a = jnp.minimum(a, v)
                                    accs[k][pl.ds(off + jj * 16, 16)] = a
                    return c2

                lax.fori_loop(0, chunk // 16, grp_body, 0)
            return c

        lax.fori_loop(0, n_it, body, 0)
        for k in range(len(ops)):
            pltpu.sync_copy(
                accs[k], outs[k].at[pl.ds(wid * nseg * width, nseg * width)])

    return pk


def _sc_pools(table, seg_pad, n_real, nseg, ops, chunk=256):
    r_pad, width = table.shape
    outs = _build_pools(r_pad, n_real, width, nseg, chunk, tuple(ops))(
        table.reshape(-1), seg_pad)
    return [o.reshape(NW, nseg, width) for o in outs]


@functools.cache
def _build_linear2(n, din, dout, nb, has_scale):
    grid = n // nb

    def body(*refs):
        if has_scale:
            x_ref, s_ref, w_ref, a_ref, b_ref = refs
            xv = x_ref[...] * s_ref[...]
        else:
            x_ref, w_ref, a_ref, b_ref = refs
            xv = x_ref[...]
        o = jnp.dot(xv, w_ref[...], preferred_element_type=F32)
        a_ref[...] = o[:, :dout]
        b_ref[...] = o[:, dout:]

    in_specs = [pl.BlockSpec((nb, din), lambda i: (i, 0))]
    if has_scale:
        in_specs.append(pl.BlockSpec((nb, 1), lambda i: (i, 0)))
    in_specs.append(pl.BlockSpec((din, 2 * dout), lambda i: (0, 0)))
    return pl.pallas_call(
        body, grid=(grid,),
        in_specs=in_specs,
        out_specs=[pl.BlockSpec((nb, dout), lambda i: (i, 0))] * 2,
        out_shape=[jax.ShapeDtypeStruct((n, dout), F32)] * 2,
    )


def _tc_proj2(x, scale, w_ab, nb=1000):
    n, din = x.shape
    dout = w_ab.shape[1] // 2
    if scale is None:
        return _build_linear2(n, din, dout, nb, False)(x, w_ab)
    return _build_linear2(n, din, dout, nb, True)(x, scale, w_ab)


@functools.cache
def _build_mlp2(n, din, dhid, dout, nb, has_scale):
    def body(*refs):
        if has_scale:
            x_ref, s_ref, w1, b1, w2, b2, o_ref = refs
            xv = x_ref[...] * s_ref[...]
        else:
            x_ref, w1, b1, w2, b2, o_ref = refs
            xv = x_ref[...]
        h = jnp.dot(xv, w1[...], preferred_element_type=F32) + b1[...]
        h = jnp.maximum(h, 0.0)
        o = jnp.dot(h, w2[...], preferred_element_type=F32) + b2[...]
        o_ref[...] = jnp.maximum(o, 0.0)

    grid = n // nb
    in_specs = [pl.BlockSpec((nb, din), lambda i: (i, 0))]
    if has_scale:
        in_specs.append(pl.BlockSpec((nb, 1), lambda i: (i, 0)))
    in_specs += [
        pl.BlockSpec((din, dhid), lambda i: (0, 0)),
        pl.BlockSpec((1, dhid), lambda i: (0, 0)),
        pl.BlockSpec((dhid, dout), lambda i: (0, 0)),
        pl.BlockSpec((1, dout), lambda i: (0, 0)),
    ]
    return pl.pallas_call(
        body, grid=(grid,),
        in_specs=in_specs,
        out_specs=pl.BlockSpec((nb, dout), lambda i: (i, 0)),
        out_shape=jax.ShapeDtypeStruct((n, dout), F32),
    )


def _tc_mlp2(x, scale, w1, b1, w2, b2, nb=1000):
    n, din = x.shape
    dhid, dout = w1.shape[1], w2.shape[1]
    args = (x,) if scale is None else (x, scale)
    return _build_mlp2(n, din, dhid, dout, nb, scale is not None)(
        *args, w1, b1.reshape(1, -1), w2, b2.reshape(1, -1))


@functools.cache
def _build_edge_feat(e_pad, f, eb):
    def body(xr_ref, xs_ref, ep_ref, vz_ref):
        diff = xr_ref[...] - xs_ref[...]
        d3 = diff[:, :3]
        ss = jnp.sum(d3 * d3, axis=1, keepdims=True)
        zero = ss == 0.0
        dist = jnp.where(zero, 0.0, jnp.sqrt(jnp.where(zero, 1.0, ss)))
        dsafe = jnp.where(dist == 0.0, 1.0, dist)
        vects = jnp.where(dist == 0.0, 0.0, d3 / dsafe)
        ep_ref[...] = jnp.concatenate([diff[:, 3:], dist, vects[:, :2]],
                                      axis=1)
        vz_ref[...] = vects[:, 2:3]

    grid = e_pad // eb
    return pl.pallas_call(
        body, grid=(grid,),
        in_specs=[pl.BlockSpec((eb, f), lambda i: (i, 0))] * 2,
        out_specs=[pl.BlockSpec((eb, f), lambda i: (i, 0)),
                   pl.BlockSpec((eb, 1), lambda i: (i, 0))],
        out_shape=[jax.ShapeDtypeStruct((e_pad, f), F32),
                   jax.ShapeDtypeStruct((e_pad, 1), F32)],
    )


@functools.cache
def _build_edge_msg(e_pad, dhid, dout, eb):
    def body(ai, bj, ep, vz, w1c, wz, b1, w2, b2, m_ref):
        h = ai[...] + bj[...]
        h = h + jnp.dot(ep[...], w1c[...], preferred_element_type=F32)
        h = h + vz[...] * wz[...] + b1[...]
        h = jnp.maximum(h, 0.0)
        m = jnp.dot(h, w2[...], preferred_element_type=F32) + b2[...]
        m_ref[...] = jnp.maximum(m, 0.0)

    grid = e_pad // eb
    return pl.pallas_call(
        body, grid=(grid,),
        in_specs=[
            pl.BlockSpec((eb, dhid), lambda i: (i, 0)),
            pl.BlockSpec((eb, dhid), lambda i: (i, 0)),
            pl.BlockSpec((eb, 128), lambda i: (i, 0)),
            pl.BlockSpec((eb, 1), lambda i: (i, 0)),
            pl.BlockSpec((128, dhid), lambda i: (0, 0)),
            pl.BlockSpec((1, dhid), lambda i: (0, 0)),
            pl.BlockSpec((1, dhid), lambda i: (0, 0)),
            pl.BlockSpec((dhid, dout), lambda i: (0, 0)),
            pl.BlockSpec((1, dout), lambda i: (0, 0)),
        ],
        out_specs=pl.BlockSpec((eb, dout), lambda i: (i, 0)),
        out_shape=jax.ShapeDtypeStruct((e_pad, dout), F32),
    )


def _tc_edge_msg(ai, bj, ep, vz, w1c, wz, b1, w2, b2, eb=1024):
    e_pad, dhid = ai.shape
    dout = w2.shape[1]
    return _build_edge_msg(e_pad, dhid, dout, eb)(
        ai, bj, ep, vz, w1c, wz.reshape(1, -1), b1.reshape(1, -1), w2,
        b2.reshape(1, -1))


@functools.cache
def _build_sage(n, din, dout, nb):
    def body(h, ns, inv, wt, wb, b, o_ref):
        o = jnp.dot(h[...], wt[...], preferred_element_type=F32)
        o = o + jnp.dot(ns[...] * inv[...], wb[...],
                        preferred_element_type=F32)
        o = o + b[...]
        nrm = jnp.maximum(jnp.sum(o * o, axis=-1, keepdims=True), 1e-12)
        o = o * lax.rsqrt(nrm)
        o_ref[...] = jnp.maximum(o, 0.0)

    grid = n // nb
    return pl.pallas_call(
        body, grid=(grid,),
        in_specs=[
            pl.BlockSpec((nb, din), lambda i: (i, 0)),
            pl.BlockSpec((nb, din), lambda i: (i, 0)),
            pl.BlockSpec((nb, 1), lambda i: (i, 0)),
            pl.BlockSpec((din, dout), lambda i: (0, 0)),
            pl.BlockSpec((din, dout), lambda i: (0, 0)),
            pl.BlockSpec((1, dout), lambda i: (0, 0)),
        ],
        out_specs=pl.BlockSpec((nb, dout), lambda i: (i, 0)),
        out_shape=jax.ShapeDtypeStruct((n, dout), F32),
    )


def _tc_sage(h, nsum, inv_deg, w, b, nb=1000):
    n, din = h.shape
    dout = w.shape[1]
    return _build_sage(n, din, dout, nb)(
        h, nsum, inv_deg, w[:din], w[din:], b.reshape(1, -1))


@functools.cache
def _build_pool_combine(g, wh, wx):
    def body(hmax, hsum, xsum, xsq, xmax, xmin, cnt, z_ref):
        c = jnp.maximum(cnt[...], 1.0)
        p1 = jnp.max(hmax[...], axis=0)
        p3 = jnp.sum(hsum[...], axis=0)
        p2 = p3 / c
        xs = jnp.sum(xsum[...], axis=0)
        avg = xs / c
        sq = jnp.sum(xsq[...], axis=0) / c
        var = jnp.abs(sq - avg * avg)
        xma = jnp.max(xmax[...], axis=0)
        xmi = jnp.min(xmin[...], axis=0)
        z_ref[...] = jnp.concatenate([p1, p2, p3, avg, var, xma, xmi], axis=1)

    return pl.pallas_call(
        body,
        in_specs=[
            pl.BlockSpec((NW, g, wh), lambda: (0, 0, 0)),
            pl.BlockSpec((NW, g, wh), lambda: (0, 0, 0)),
            pl.BlockSpec((NW, g, wx), lambda: (0, 0, 0)),
            pl.BlockSpec((NW, g, wx), lambda: (0, 0, 0)),
            pl.BlockSpec((NW, g, wx), lambda: (0, 0, 0)),
            pl.BlockSpec((NW, g, wx), lambda: (0, 0, 0)),
            pl.BlockSpec((g, 1), lambda: (0, 0)),
        ],
        out_specs=pl.BlockSpec((g, 3 * wh + 4 * wx), lambda: (0, 0)),
        out_shape=jax.ShapeDtypeStruct((g, 3 * wh + 4 * wx), F32),
    )


@functools.cache
def _build_declayer(g, din, dout):
    def body(z, w, b, s, t, o_ref):
        u = jnp.dot(z[...], w[...], preferred_element_type=F32) + b[...]
        u = jnp.where(u >= 0.0, u, 0.15 * u)
        o_ref[...] = s[...] * u + t[...]

    return pl.pallas_call(
        body,
        in_specs=[
            pl.BlockSpec((g, din), lambda: (0, 0)),
            pl.BlockSpec((din, dout), lambda: (0, 0)),
            pl.BlockSpec((1, dout), lambda: (0, 0)),
            pl.BlockSpec((1, dout), lambda: (0, 0)),
            pl.BlockSpec((1, dout), lambda: (0, 0)),
        ],
        out_specs=pl.BlockSpec((g, dout), lambda: (0, 0)),
        out_shape=jax.ShapeDtypeStruct((g, dout), F32),
    )


@functools.cache
def _build_heads(g, din, h):
    def body(z, wl0, bl0, wl1, bl1, wl2, bl2, wa0, ba0, wa1, ba1, wa2, ba2,
             was, bas, ws0, bs0, ws1, bs1, ws2, bs2, o_ref):
        zv = z[...]

        def lin(v, w, b):
            return jnp.dot(v, w[...], preferred_element_type=F32) + b[...]

        lg = lin(lin(lin(zv, wl0, bl0), wl1, bl1), wl2, bl2)
        an = lin(lin(lin(zv, wa0, ba0), wa1, ba1), wa2, ba2)
        za = 1.0 / (1.0 + jnp.exp(-lin(an, was, bas)))
        sg = jnp.abs(lin(lin(lin(zv, ws0, bs0), ws1, bs1), ws2, bs2)) + _EPS
        xs = jnp.concatenate(
            [lg[:, :1], za[:, :1] * np.pi, za[:, 1:2] * (2.0 * np.pi)], axis=1)
        o_ref[...] = jnp.concatenate([xs, sg], axis=1)

    def spec(a, b):
        return pl.BlockSpec((a, b), lambda: (0, 0))

    return pl.pallas_call(
        body,
        in_specs=[
            spec(g, din),
            spec(din, h), spec(1, h), spec(h, h), spec(1, h), spec(h, 1),
            spec(1, 1),
            spec(din, h), spec(1, h), spec(h, h), spec(1, h), spec(h, 2),
            spec(1, 2),
            spec(2, 2), spec(1, 2),
            spec(din, h), spec(1, h), spec(h, h), spec(1, h), spec(h, 2),
            spec(1, 2),
        ],
        out_specs=pl.BlockSpec((g, 5), lambda: (0, 0)),
        out_shape=jax.ShapeDtypeStruct((g, 5), F32),
    )


def _bn_affine(p):
    s = p["gamma"] / jnp.sqrt(p["var"] + 1e-3)
    t = p["beta"] - p["mean"] * s
    return s, t


def _prep_hop(hp, d, bn_s, bn_t):
    w1, b1 = hp["W1"], hp["b1"]
    w_ab = jnp.concatenate([w1[:d], w1[d:2 * d]], axis=1)
    w1e = bn_s[:, None] * w1[2 * d:]
    b1_eff = b1 + bn_t @ w1[2 * d:]
    return w_ab, w1e[:128], w1e[128], b1_eff, hp["W2"], hp["b2"]


def kernel(x, params, edge_index, seg_i):
    n, f = x.shape
    e = edge_index.shape[0]
    e_pad = e + EPAD_EXTRA
    g = 16
    npw = 313

    send = edge_index[:, 0].astype(I32)
    recv = edge_index[:, 1].astype(I32)
    perm = jnp.argsort(recv)
    recv_s = recv[perm]
    send_s = send[perm]
    pad_i = jnp.zeros((EPAD_EXTRA,), I32)
    ridx = jnp.concatenate([recv_s, pad_i])
    sidx = jnp.concatenate([send_s, pad_i])
    seg_pad = jnp.concatenate([recv_s, jnp.full((EPAD_EXTRA,), n, I32)])

    node_pos = jnp.searchsorted(recv_s, jnp.arange(n + 1), side="left")
    deg = (node_pos[1:] - node_pos[:-1]).astype(F32)
    inv_deg = (1.0 / jnp.maximum(deg, 1.0)).reshape(n, 1)

    def seg_bounds(node_base, npw_):
        tgt = node_base + jnp.arange(NW + 1) * npw_
        b = jnp.searchsorted(recv_s, jnp.clip(tgt, 0, n), side="left")
        return jnp.concatenate(
            [b, jnp.array([node_base], b.dtype),
             jnp.zeros((40 - (NW + 2),), b.dtype)]).astype(I32)

    bounds64 = seg_bounds(0, npw)

    xr, xs = _sc_gather2(x, x, ridx, sidx)
    ep, vz = _build_edge_feat(e_pad, f, 1024)(xr, xs)
    bn_s, bn_t = _bn_affine(params["norm_edge"])

    def run_conv(xa, scale, cp, method):
        op = "sum" if method == "mean" else method
        for hp in cp["hops"]:
            d = xa.shape[1]
            w_ab, w1c, wz, b1e, w2, b2 = _prep_hop(hp, d, bn_s, bn_t)
            a_t, b_t = _tc_proj2(xa, scale, w_ab)
            ai, bj = _sc_gather2(a_t, b_t, ridx, sidx)
            m = _tc_edge_msg(ai, bj, ep, vz, w1c, wz, b1e, w2, b2)
            xa = _sc_segreduce(m, seg_pad, bounds64, n, 64, npw, 512, op)
            scale = inv_deg if method == "mean" else None
        u = cp["update"]
        return _tc_mlp2(xa, scale, u["W1"], u["b1"], u["W2"], u["b2"])

    x2me = run_conv(x, None, params["hop2mean"], "mean")
    x12mi = run_conv(x, None, params["hop12min1"], "min")
    x12mi = run_conv(x12mi, None, params["hop12min2"], "min")
    x12ma = run_conv(x, None, params["hop12max1"], "max")
    x12ma = run_conv(x12ma, None, params["hop12max2"], "max")
    x23mi = run_conv(x, None, params["hop23min2"], "min")
    x23mi = run_conv(x23mi, None, params["hop23min3"], "min")

    hcat = jnp.concatenate([x, x2me, x12mi, x12ma, x23mi], axis=1)

    hs = _sc_gather(hcat, sidx, chunk=128, nbuf=2)
    half = NW * 157
    ns_lo = _sc_segreduce(hs, seg_pad, seg_bounds(0, 157), half, 384, 157,
                          128, "sum")
    ns_hi = _sc_segreduce(hs, seg_pad, seg_bounds(half, 157), n - half, 384,
                          157, 128, "sum")
    ns1 = jnp.concatenate([ns_lo, ns_hi], axis=0)
    h1 = _tc_sage(hcat, ns1, inv_deg, params["gcn1"]["W"], params["gcn1"]["b"])

    hs2 = _sc_gather(h1, sidx)
    ns2 = _sc_segreduce(hs2, seg_pad, bounds64, n, 128, npw, 256, "sum")
    h2 = _tc_sage(h1, ns2, inv_deg, params["gcn2"]["W"], params["gcn2"]["b"])

    r_pad = 10240
    segp = jnp.concatenate(
        [seg_i.astype(I32), jnp.zeros((r_pad - n,), I32)])
    xp = jnp.concatenate([x, jnp.zeros((r_pad - n, f), F32)], axis=0)
    h2p = jnp.concatenate(
        [h2, jnp.zeros((r_pad - n, h2.shape[1]), F32)], axis=0)
    hmax, hsum = _sc_pools(h2p, segp, n, g, ("max", "sum"))
    xsum, xsq, xmax, xmin = _sc_pools(xp, segp, n, g,
                                      ("sum", "sumsq", "max", "min"))
    gpos = jnp.searchsorted(seg_i, jnp.arange(g + 1), side="left")
    cnt = (gpos[1:] - gpos[:-1]).astype(F32).reshape(g, 1)

    z = _build_pool_combine(g, h2.shape[1], f)(
        hmax, hsum, xsum, xsq, xmax, xmin, cnt)

    for dp in params["decode"]:
        s, t = _bn_affine(dp["bn"])
        z = _build_declayer(g, z.shape[1], dp["W"].shape[1])(
            z, dp["W"], dp["b"].reshape(1, -1), s.reshape(1, -1),
            t.reshape(1, -1))

    p = params
    return _build_heads(g, z.shape[1], 64)(
        z,
        p["loge0"]["W"], p["loge0"]["b"].reshape(1, -1),
        p["loge1"]["W"], p["loge1"]["b"].reshape(1, -1),
        p["loge_out"]["W"], p["loge_out"]["b"].reshape(1, -1),
        p["angles0"]["W"], p["angles0"]["b"].reshape(1, -1),
        p["angles1"]["W"], p["angles1"]["b"].reshape(1, -1),
        p["angles_out"]["W"], p["angles_out"]["b"].reshape(1, -1),
        p["angle_scale"]["W"], p["angle_scale"]["b"].reshape(1, -1),
        p["sigs0"]["W"], p["sigs0"]["b"].reshape(1, -1),
        p["sigs1"]["W"], p["sigs1"]["b"].reshape(1, -1),
        p["sigs_out"]["W"], p["sigs_out"]["b"].reshape(1, -1),
    )


# speedup vs baseline: 1.0895x; 1.0895x over previous
import functools
import math

import numpy as np
import jax
import jax.numpy as jnp
from jax import lax
from jax.experimental import pallas as pl
from jax.experimental.pallas import tpu as pltpu
from jax.experimental.pallas import tpu_sc as plsc

F32 = jnp.float32
I32 = jnp.int32
NW = 32
EPAD_EXTRA = 512
_EPS = 1e-05


def _sc_mesh():
    return plsc.VectorSubcoreMesh(core_axis_name="c", subcore_axis_name="s")


def _wid():
    return lax.axis_index("s") * 2 + lax.axis_index("c")


@functools.cache
def _build_gather(n_rows_out, d, chunk, mode, nbuf):
    two = mode != "copy"
    n_chunks = n_rows_out // chunk
    assert n_rows_out % chunk == 0 and d % 16 == 0 and chunk % 8 == 0
    n_it = (n_chunks + NW - 1) // NW

    out_t = jax.ShapeDtypeStruct((n_rows_out, d), F32)
    scratch = ([pltpu.VMEM((chunk,), I32) for _ in range(nbuf)]
               + [pltpu.VMEM((chunk, d), F32) for _ in range(nbuf)]
               + [pltpu.SemaphoreType.DMA for _ in range(nbuf)])
    if two:
        scratch += ([pltpu.VMEM((chunk,), I32) for _ in range(nbuf)]
                    + [pltpu.VMEM((chunk, d), F32) for _ in range(nbuf)])

    @functools.partial(
        pl.kernel, mesh=_sc_mesh(),
        out_type=out_t,
        scratch_types=scratch,
    )
    def gk(*refs):
        if two:
            (ta, tb, ia_hbm, ib_hbm, outa) = refs[:5]
            rest = refs[5:]
        else:
            ta, ia_hbm, outa = refs[:3]
            rest = refs[3:]
        ias = rest[:nbuf]
        ras = rest[nbuf:2 * nbuf]
        sems = rest[2 * nbuf:3 * nbuf]
        if two:
            ibs = rest[3 * nbuf:4 * nbuf]
            rbs = rest[4 * nbuf:5 * nbuf]
        wid = _wid()

        def start(it, b):
            cid = it * NW + wid

            @pl.when(cid < n_chunks)
            def _():
                base = cid * chunk
                pltpu.sync_copy(ia_hbm.at[pl.ds(base, chunk)], ias[b])
                pltpu.async_copy(ta.at[ias[b]], ras[b], sems[b])
                if two:
                    pltpu.sync_copy(ib_hbm.at[pl.ds(base, chunk)], ibs[b])
                    pltpu.async_copy(tb.at[ibs[b]], rbs[b], sems[b])

        def finish(it, b):
            cid = it * NW + wid

            @pl.when(cid < n_chunks)
            def _():
                pltpu.make_async_copy(ta.at[ias[b]], ras[b], sems[b]).wait()
                if two:
                    pltpu.make_async_copy(tb.at[ibs[b]], rbs[b],
                                          sems[b]).wait()

                    def addb(i, c):
                        for jj in range(d // 16):
                            va = ras[b][i, pl.ds(jj * 16, 16)]
                            vb = rbs[b][i, pl.ds(jj * 16, 16)]
                            ras[b][i, pl.ds(jj * 16, 16)] = (
                                va + vb if mode == "add" else va - vb)
                        return c

                    lax.fori_loop(0, chunk, addb, 0)
                pltpu.sync_copy(ras[b], outa.at[pl.ds(cid * chunk, chunk)])

        for b in range(nbuf):
            start(b, b)
        n_grp = max((n_it - nbuf) // nbuf, 0)

        def grp(g, c):
            for b in range(nbuf):
                it = nbuf * g + b
                finish(it, b)
                start(it + nbuf, b)
            return c

        lax.fori_loop(0, n_grp, grp, 0)
        for it in range(nbuf * n_grp, n_it):
            finish(it, it % nbuf)
            start(it + nbuf, it % nbuf)

    return gk


def _sc_gather(table, idx, chunk=256, nbuf=3):
    table = jnp.asarray(table, F32)
    return _build_gather(idx.shape[0], table.shape[1], chunk, "copy", nbuf)(
        table, idx)


def _sc_gather2(mode, ta, tb, ia, ib, chunk=128, nbuf=3):
    return _build_gather(ia.shape[0], ta.shape[1], chunk, mode, nbuf)(
        ta, tb, ia, ib)


@functools.cache
def _build_segreduce(e_pad, width, npw, chunk, op):
    assert width % 16 == 0 and chunk % 8 == 0
    nvec = width // 16
    ident = {"sum": 0.0, "min": float("inf"), "max": float("-inf")}[op]
    red = {"sum": lambda a, b: a + b,
           "min": jnp.minimum, "max": jnp.maximum}[op]

    @functools.partial(
        pl.kernel, mesh=_sc_mesh(),
        out_type=jax.ShapeDtypeStruct((NW * npw * width,), F32),
        scratch_types=[
            pltpu.VMEM((56,), I32),
            pltpu.VMEM((chunk + 16,), I32),
            pltpu.VMEM((npw * width,), F32),
            pltpu.VMEM((chunk * width + 16 * nvec,), F32),
        ],
    )
    def srk(msgs_hbm, seg_hbm, bounds_hbm, out_hbm, bounds_s, seg_s,
            acc_v, chunk_v):
        wid = _wid()
        pltpu.sync_copy(bounds_hbm, bounds_s.at[pl.ds(0, 40)])

        def initb(i, c):
            acc_v[pl.ds(i * 16, 16)] = jnp.full((16,), ident, F32)
            return c

        lax.fori_loop(0, npw * width // 16, initb, 0)
        ident_base = chunk * width
        for jj in range(nvec):
            chunk_v[pl.ds(ident_base + jj * 16, 16)] = jnp.full(
                (16,), ident, F32)

        start = bounds_s[pl.ds(wid, 16)][0]
        end = bounds_s[pl.ds(wid + 1, 16)][0]
        node_base = bounds_s[pl.ds(33, 16)][0]
        abase = (start // 8) * 8
        nch = (end - abase + chunk - 1) // chunk
        base0 = node_base + wid * npw

        def chunk_body(ic, c):
            base = abase + ic * chunk
            pltpu.sync_copy(msgs_hbm.at[pl.ds(base * width, chunk * width)],
                            chunk_v.at[pl.ds(0, chunk * width)])
            pltpu.sync_copy(seg_hbm.at[pl.ds(base, chunk)],
                            seg_s.at[pl.ds(0, chunk)])

            def grp_body(gi, c2):
                segv = seg_s[pl.ds(gi * 16, 16)] - base0
                for l in range(16):
                    j = gi * 16 + l
                    e = base + j
                    node = jnp.clip(segv[l], 0, npw - 1)
                    valid = jnp.logical_and(e >= start, e < end)
                    off = node * width
                    moff = jnp.where(valid, j * width, ident_base)
                    for jj in range(nvec):
                        a = acc_v[pl.ds(off + jj * 16, 16)]
                        m = chunk_v[pl.ds(moff + jj * 16, 16)]
                        acc_v[pl.ds(off + jj * 16, 16)] = red(a, m)
                return c2

            lax.fori_loop(0, chunk // 16, grp_body, 0)
            return c

        lax.fori_loop(0, nch, chunk_body, 0)
        pltpu.sync_copy(acc_v, out_hbm.at[pl.ds(wid * npw * width, npw * width)])

    return srk


def _sc_segreduce(msgs2d, seg_pad, bounds, n_out, width, npw, chunk, op):
    e_pad = seg_pad.shape[0]
    flat = msgs2d.reshape(-1)
    out = _build_segreduce(e_pad, width, npw, chunk, op)(flat, seg_pad, bounds)
    return out.reshape(NW * npw, width)[:n_out]


@functools.cache
def _build_pools(r_pad, n_real, width, nseg, chunk, ops):
    n_chunks = r_pad // chunk
    assert r_pad % chunk == 0 and width % 16 == 0
    n_it = (n_chunks + NW - 1) // NW
    nvec = width // 16
    idents = {"sum": 0.0, "sumsq": 0.0, "min": float("inf"),
              "max": float("-inf")}

    @functools.partial(
        pl.kernel, mesh=_sc_mesh(),
        out_type=[jax.ShapeDtypeStruct((NW * nseg * width,), F32)
                  for _ in ops],
        scratch_types=[
            pltpu.VMEM((chunk + 16,), I32),
            pltpu.VMEM((chunk * width,), F32),
        ] + [pltpu.VMEM((nseg * width,), F32) for _ in ops],
    )
    def pk(table_hbm, seg_hbm, *rest):
        outs = rest[:len(ops)]
        seg_s = rest[len(ops)]
        chunk_v = rest[len(ops) + 1]
        accs = rest[len(ops) + 2:]
        wid = _wid()

        def initb(i, c):
            for k, o in enumerate(ops):
                accs[k][pl.ds(i * 16, 16)] = jnp.full((16,), idents[o], F32)
            return c

        lax.fori_loop(0, nseg * width // 16, initb, 0)

        def body(it, c):
            cid = it * NW + wid

            @pl.when(cid < n_chunks)
            def _():
                base = cid * chunk
                pltpu.sync_copy(
                    table_hbm.at[pl.ds(base * width, chunk * width)], chunk_v)
                pltpu.sync_copy(seg_hbm.at[pl.ds(base, chunk)],
                                seg_s.at[pl.ds(0, chunk)])

                def grp_body(gi, c2):
                    segv = seg_s[pl.ds(gi * 16, 16)]
                    for l in range(16):
                        j = gi * 16 + l

                        @pl.when(base + j < n_real)
                        def _(l=l, j=j):
                            off = segv[l] * width
                            moff = j * width
                            for jj in range(nvec):
                                v = chunk_v[pl.ds(moff + jj * 16, 16)]
                                for k, o in enumerate(ops):
                                    a = accs[k][pl.ds(off + jj * 16, 16)]
                                    if o == "sum":
                                        a = a + v
                                    elif o == "sumsq":
                                        a = a + v * v
                                    elif o == "max":
                                        a = jnp.maximum(a, v)
                                    else:
                                        a = jnp.minimum(a, v)
                                    accs[k][pl.ds(off + jj * 16, 16)] = a
                    return c2

                lax.fori_loop(0, chunk // 16, grp_body, 0)
            return c

        lax.fori_loop(0, n_it, body, 0)
        for k in range(len(ops)):
            pltpu.sync_copy(
                accs[k], outs[k].at[pl.ds(wid * nseg * width, nseg * width)])

    return pk


def _sc_pools(table, seg_pad, n_real, nseg, ops, chunk=256):
    r_pad, width = table.shape
    outs = _build_pools(r_pad, n_real, width, nseg, chunk, tuple(ops))(
        table.reshape(-1), seg_pad)
    return [o.reshape(NW, nseg, width) for o in outs]


@functools.cache
def _build_linear2(n, din, dout, nb, has_scale):
    grid = n // nb

    def body(*refs):
        if has_scale:
            x_ref, s_ref, w_ref, a_ref, b_ref = refs
            xv = x_ref[...] * s_ref[...]
        else:
            x_ref, w_ref, a_ref, b_ref = refs
            xv = x_ref[...]
        o = jnp.dot(xv, w_ref[...], preferred_element_type=F32)
        a_ref[...] = o[:, :dout]
        b_ref[...] = o[:, dout:]

    in_specs = [pl.BlockSpec((nb, din), lambda i: (i, 0))]
    if has_scale:
        in_specs.append(pl.BlockSpec((nb, 1), lambda i: (i, 0)))
    in_specs.append(pl.BlockSpec((din, 2 * dout), lambda i: (0, 0)))
    return pl.pallas_call(
        body, grid=(grid,),
        in_specs=in_specs,
        out_specs=[pl.BlockSpec((nb, dout), lambda i: (i, 0))] * 2,
        out_shape=[jax.ShapeDtypeStruct((n, dout), F32)] * 2,
    )


def _tc_proj2(x, scale, w_ab, nb=1000):
    n, din = x.shape
    dout = w_ab.shape[1] // 2
    if scale is None:
        return _build_linear2(n, din, dout, nb, False)(x, w_ab)
    return _build_linear2(n, din, dout, nb, True)(x, scale, w_ab)


@functools.cache
def _build_mlp2(n, din, dhid, dout, nb, has_scale):
    def body(*refs):
        if has_scale:
            x_ref, s_ref, w1, b1, w2, b2, o_ref = refs
            xv = x_ref[...] * s_ref[...]
        else:
            x_ref, w1, b1, w2, b2, o_ref = refs
            xv = x_ref[...]
        h = jnp.dot(xv, w1[...], preferred_element_type=F32) + b1[...]
        h = jnp.maximum(h, 0.0)
        o = jnp.dot(h, w2[...], preferred_element_type=F32) + b2[...]
        o_ref[...] = jnp.maximum(o, 0.0)

    grid = n // nb
    in_specs = [pl.BlockSpec((nb, din), lambda i: (i, 0))]
    if has_scale:
        in_specs.append(pl.BlockSpec((nb, 1), lambda i: (i, 0)))
    in_specs += [
        pl.BlockSpec((din, dhid), lambda i: (0, 0)),
        pl.BlockSpec((1, dhid), lambda i: (0, 0)),
        pl.BlockSpec((dhid, dout), lambda i: (0, 0)),
        pl.BlockSpec((1, dout), lambda i: (0, 0)),
    ]
    return pl.pallas_call(
        body, grid=(grid,),
        in_specs=in_specs,
        out_specs=pl.BlockSpec((nb, dout), lambda i: (i, 0)),
        out_shape=jax.ShapeDtypeStruct((n, dout), F32),
    )


def _tc_mlp2(x, scale, w1, b1, w2, b2, nb=1000):
    n, din = x.shape
    dhid, dout = w1.shape[1], w2.shape[1]
    args = (x,) if scale is None else (x, scale)
    return _build_mlp2(n, din, dhid, dout, nb, scale is not None)(
        *args, w1, b1.reshape(1, -1), w2, b2.reshape(1, -1))


@functools.cache
def _build_edge_feat(e_pad, f, eb):
    def body(diff_ref, ep_ref, vz_ref):
        diff = diff_ref[...]
        d3 = diff[:, :3]
        ss = jnp.sum(d3 * d3, axis=1, keepdims=True)
        zero = ss == 0.0
        dist = jnp.where(zero, 0.0, jnp.sqrt(jnp.where(zero, 1.0, ss)))
        dsafe = jnp.where(dist == 0.0, 1.0, dist)
        vects = jnp.where(dist == 0.0, 0.0, d3 / dsafe)
        ep_ref[...] = jnp.concatenate([diff[:, 3:], dist, vects[:, :2]],
                                      axis=1)
        vz_ref[...] = vects[:, 2:3]

    grid = e_pad // eb
    return pl.pallas_call(
        body, grid=(grid,),
        in_specs=[pl.BlockSpec((eb, f), lambda i: (i, 0))],
        out_specs=[pl.BlockSpec((eb, f), lambda i: (i, 0)),
                   pl.BlockSpec((eb, 1), lambda i: (i, 0))],
        out_shape=[jax.ShapeDtypeStruct((e_pad, f), F32),
                   jax.ShapeDtypeStruct((e_pad, 1), F32)],
    )


@functools.cache
def _build_edge_msg(e_pad, dhid, dout, eb):
    def body(s, ep, vz, w1c, wz, b1, w2, b2, m_ref):
        h = s[...]
        h = h + jnp.dot(ep[...], w1c[...], preferred_element_type=F32)
        h = h + vz[...] * wz[...] + b1[...]
        h = jnp.maximum(h, 0.0)
        m = jnp.dot(h, w2[...], preferred_element_type=F32) + b2[...]
        m_ref[...] = jnp.maximum(m, 0.0)

    grid = e_pad // eb
    return pl.pallas_call(
        body, grid=(grid,),
        in_specs=[
            pl.BlockSpec((eb, dhid), lambda i: (i, 0)),
            pl.BlockSpec((eb, 128), lambda i: (i, 0)),
            pl.BlockSpec((eb, 1), lambda i: (i, 0)),
            pl.BlockSpec((128, dhid), lambda i: (0, 0)),
            pl.BlockSpec((1, dhid), lambda i: (0, 0)),
            pl.BlockSpec((1, dhid), lambda i: (0, 0)),
            pl.BlockSpec((dhid, dout), lambda i: (0, 0)),
            pl.BlockSpec((1, dout), lambda i: (0, 0)),
        ],
        out_specs=pl.BlockSpec((eb, dout), lambda i: (i, 0)),
        out_shape=jax.ShapeDtypeStruct((e_pad, dout), F32),
    )


def _tc_edge_msg(s, ep, vz, w1c, wz, b1, w2, b2, eb=1024):
    e_pad, dhid = s.shape
    dout = w2.shape[1]
    return _build_edge_msg(e_pad, dhid, dout, eb)(
        s, ep, vz, w1c, wz.reshape(1, -1), b1.reshape(1, -1), w2,
        b2.reshape(1, -1))


@functools.cache
def _build_sage(n, din, dout, nb):
    def body(h, ns, inv, wt, wb, b, o_ref):
        o = jnp.dot(h[...], wt[...], preferred_element_type=F32)
        o = o + jnp.dot(ns[...] * inv[...], wb[...],
                        preferred_element_type=F32)
        o = o + b[...]
        nrm = jnp.maximum(jnp.sum(o * o, axis=-1, keepdims=True), 1e-12)
        o = o * lax.rsqrt(nrm)
        o_ref[...] = jnp.maximum(o, 0.0)

    grid = n // nb
    return pl.pallas_call(
        body, grid=(grid,),
        in_specs=[
            pl.BlockSpec((nb, din), lambda i: (i, 0)),
            pl.BlockSpec((nb, din), lambda i: (i, 0)),
            pl.BlockSpec((nb, 1), lambda i: (i, 0)),
            pl.BlockSpec((din, dout), lambda i: (0, 0)),
            pl.BlockSpec((din, dout), lambda i: (0, 0)),
            pl.BlockSpec((1, dout), lambda i: (0, 0)),
        ],
        out_specs=pl.BlockSpec((nb, dout), lambda i: (i, 0)),
        out_shape=jax.ShapeDtypeStruct((n, dout), F32),
    )


def _tc_sage(h, nsum, inv_deg, w, b, nb=1000):
    n, din = h.shape
    dout = w.shape[1]
    return _build_sage(n, din, dout, nb)(
        h, nsum, inv_deg, w[:din], w[din:], b.reshape(1, -1))


@functools.cache
def _build_pool_combine(g, wh, wx):
    def body(hmax, hsum, xsum, xsq, xmax, xmin, cnt, z_ref):
        c = jnp.maximum(cnt[...], 1.0)
        p1 = jnp.max(hmax[...], axis=0)
        p3 = jnp.sum(hsum[...], axis=0)
        p2 = p3 / c
        xs = jnp.sum(xsum[...], axis=0)
        avg = xs / c
        sq = jnp.sum(xsq[...], axis=0) / c
        var = jnp.abs(sq - avg * avg)
        xma = jnp.max(xmax[...], axis=0)
        xmi = jnp.min(xmin[...], axis=0)
        z_ref[...] = jnp.concatenate([p1, p2, p3, avg, var, xma, xmi], axis=1)

    return pl.pallas_call(
        body,
        in_specs=[
            pl.BlockSpec((NW, g, wh), lambda: (0, 0, 0)),
            pl.BlockSpec((NW, g, wh), lambda: (0, 0, 0)),
            pl.BlockSpec((NW, g, wx), lambda: (0, 0, 0)),
            pl.BlockSpec((NW, g, wx), lambda: (0, 0, 0)),
            pl.BlockSpec((NW, g, wx), lambda: (0, 0, 0)),
            pl.BlockSpec((NW, g, wx), lambda: (0, 0, 0)),
            pl.BlockSpec((g, 1), lambda: (0, 0)),
        ],
        out_specs=pl.BlockSpec((g, 3 * wh + 4 * wx), lambda: (0, 0)),
        out_shape=jax.ShapeDtypeStruct((g, 3 * wh + 4 * wx), F32),
    )


@functools.cache
def _build_declayer(g, din, dout):
    def body(z, w, b, s, t, o_ref):
        u = jnp.dot(z[...], w[...], preferred_element_type=F32) + b[...]
        u = jnp.where(u >= 0.0, u, 0.15 * u)
        o_ref[...] = s[...] * u + t[...]

    return pl.pallas_call(
        body,
        in_specs=[
            pl.BlockSpec((g, din), lambda: (0, 0)),
            pl.BlockSpec((din, dout), lambda: (0, 0)),
            pl.BlockSpec((1, dout), lambda: (0, 0)),
            pl.BlockSpec((1, dout), lambda: (0, 0)),
            pl.BlockSpec((1, dout), lambda: (0, 0)),
        ],
        out_specs=pl.BlockSpec((g, dout), lambda: (0, 0)),
        out_shape=jax.ShapeDtypeStruct((g, dout), F32),
    )


@functools.cache
def _build_heads(g, din, h):
    def body(z, wl0, bl0, wl1, bl1, wl2, bl2, wa0, ba0, wa1, ba1, wa2, ba2,
             was, bas, ws0, bs0, ws1, bs1, ws2, bs2, o_ref):
        zv = z[...]

        def lin(v, w, b):
            return jnp.dot(v, w[...], preferred_element_type=F32) + b[...]

        lg = lin(lin(lin(zv, wl0, bl0), wl1, bl1), wl2, bl2)
        an = lin(lin(lin(zv, wa0, ba0), wa1, ba1), wa2, ba2)
        za = 1.0 / (1.0 + jnp.exp(-lin(an, was, bas)))
        sg = jnp.abs(lin(lin(lin(zv, ws0, bs0), ws1, bs1), ws2, bs2)) + _EPS
        xs = jnp.concatenate(
            [lg[:, :1], za[:, :1] * np.pi, za[:, 1:2] * (2.0 * np.pi)], axis=1)
        o_ref[...] = jnp.concatenate([xs, sg], axis=1)

    def spec(a, b):
        return pl.BlockSpec((a, b), lambda: (0, 0))

    return pl.pallas_call(
        body,
        in_specs=[
            spec(g, din),
            spec(din, h), spec(1, h), spec(h, h), spec(1, h), spec(h, 1),
            spec(1, 1),
            spec(din, h), spec(1, h), spec(h, h), spec(1, h), spec(h, 2),
            spec(1, 2),
            spec(2, 2), spec(1, 2),
            spec(din, h), spec(1, h), spec(h, h), spec(1, h), spec(h, 2),
            spec(1, 2),
        ],
        out_specs=pl.BlockSpec((g, 5), lambda: (0, 0)),
        out_shape=jax.ShapeDtypeStruct((g, 5), F32),
    )


def _bn_affine(p):
    s = p["gamma"] / jnp.sqrt(p["var"] + 1e-3)
    t = p["beta"] - p["mean"] * s
    return s, t


def _prep_hop(hp, d, bn_s, bn_t):
    w1, b1 = hp["W1"], hp["b1"]
    w_ab = jnp.concatenate([w1[:d], w1[d:2 * d]], axis=1)
    w1e = bn_s[:, None] * w1[2 * d:]
    b1_eff = b1 + bn_t @ w1[2 * d:]
    return w_ab, w1e[:128], w1e[128], b1_eff, hp["W2"], hp["b2"]


def kernel(x, params, edge_index, seg_i):
    n, f = x.shape
    e = edge_index.shape[0]
    e_pad = e + EPAD_EXTRA
    g = 16
    npw = 313

    send = edge_index[:, 0].astype(I32)
    recv = edge_index[:, 1].astype(I32)
    perm = jnp.argsort(recv)
    recv_s = recv[perm]
    send_s = send[perm]
    pad_i = jnp.zeros((EPAD_EXTRA,), I32)
    ridx = jnp.concatenate([recv_s, pad_i])
    sidx = jnp.concatenate([send_s, pad_i])
    seg_pad = jnp.concatenate([recv_s, jnp.full((EPAD_EXTRA,), n, I32)])

    node_pos = jnp.searchsorted(recv_s, jnp.arange(n + 1), side="left")
    deg = (node_pos[1:] - node_pos[:-1]).astype(F32)
    inv_deg = (1.0 / jnp.maximum(deg, 1.0)).reshape(n, 1)

    def seg_bounds(node_base, npw_):
        tgt = node_base + jnp.arange(NW + 1) * npw_
        b = jnp.searchsorted(recv_s, jnp.clip(tgt, 0, n), side="left")
        return jnp.concatenate(
            [b, jnp.array([node_base], b.dtype),
             jnp.zeros((40 - (NW + 2),), b.dtype)]).astype(I32)

    bounds64 = seg_bounds(0, npw)

    diff = _sc_gather2("sub", x, x, ridx, sidx)
    ep, vz = _build_edge_feat(e_pad, f, 1024)(diff)
    bn_s, bn_t = _bn_affine(params["norm_edge"])

    def run_conv(xa, scale, cp, method):
        op = "sum" if method == "mean" else method
        for hp in cp["hops"]:
            d = xa.shape[1]
            w_ab, w1c, wz, b1e, w2, b2 = _prep_hop(hp, d, bn_s, bn_t)
            a_t, b_t = _tc_proj2(xa, scale, w_ab)
            s = _sc_gather2("add", a_t, b_t, ridx, sidx)
            m = _tc_edge_msg(s, ep, vz, w1c, wz, b1e, w2, b2)
            xa = _sc_segreduce(m, seg_pad, bounds64, n, 64, npw, 512, op)
            scale = inv_deg if method == "mean" else None
        u = cp["update"]
        return _tc_mlp2(xa, scale, u["W1"], u["b1"], u["W2"], u["b2"])

    x2me = run_conv(x, None, params["hop2mean"], "mean")
    x12mi = run_conv(x, None, params["hop12min1"], "min")
    x12mi = run_conv(x12mi, None, params["hop12min2"], "min")
    x12ma = run_conv(x, None, params["hop12max1"], "max")
    x12ma = run_conv(x12ma, None, params["hop12max2"], "max")
    x23mi = run_conv(x, None, params["hop23min2"], "min")
    x23mi = run_conv(x23mi, None, params["hop23min3"], "min")

    hcat = jnp.concatenate([x, x2me, x12mi, x12ma, x23mi], axis=1)

    hs = _sc_gather(hcat, sidx, chunk=128, nbuf=2)
    half = NW * 157
    ns_lo = _sc_segreduce(hs, seg_pad, seg_bounds(0, 157), half, 384, 157,
                          128, "sum")
    ns_hi = _sc_segreduce(hs, seg_pad, seg_bounds(half, 157), n - half, 384,
                          157, 128, "sum")
    ns1 = jnp.concatenate([ns_lo, ns_hi], axis=0)
    h1 = _tc_sage(hcat, ns1, inv_deg, params["gcn1"]["W"], params["gcn1"]["b"])

    hs2 = _sc_gather(h1, sidx)
    ns2 = _sc_segreduce(hs2, seg_pad, bounds64, n, 128, npw, 256, "sum")
    h2 = _tc_sage(h1, ns2, inv_deg, params["gcn2"]["W"], params["gcn2"]["b"])

    r_pad = 10240
    segp = jnp.concatenate(
        [seg_i.astype(I32), jnp.zeros((r_pad - n,), I32)])
    xp = jnp.concatenate([x, jnp.zeros((r_pad - n, f), F32)], axis=0)
    h2p = jnp.concatenate(
        [h2, jnp.zeros((r_pad - n, h2.shape[1]), F32)], axis=0)
    hmax, hsum = _sc_pools(h2p, segp, n, g, ("max", "sum"))
    xsum, xsq, xmax, xmin = _sc_pools(xp, segp, n, g,
                                      ("sum", "sumsq", "max", "min"))
    gpos = jnp.searchsorted(seg_i, jnp.arange(g + 1), side="left")
    cnt = (gpos[1:] - gpos[:-1]).astype(F32).reshape(g, 1)

    z = _build_pool_combine(g, h2.shape[1], f)(
        hmax, hsum, xsum, xsq, xmax, xmin, cnt)

    for dp in params["decode"]:
        s, t = _bn_affine(dp["bn"])
        z = _build_declayer(g, z.shape[1], dp["W"].shape[1])(
            z, dp["W"], dp["b"].reshape(1, -1), s.reshape(1, -1),
            t.reshape(1, -1))

    p = params
    return _build_heads(g, z.shape[1], 64)(
        z,
        p["loge0"]["W"], p["loge0"]["b"].reshape(1, -1),
        p["loge1"]["W"], p["loge1"]["b"].reshape(1, -1),
        p["loge_out"]["W"], p["loge_out"]["b"].reshape(1, -1),
        p["angles0"]["W"], p["angles0"]["b"].reshape(1, -1),
        p["angles1"]["W"], p["angles1"]["b"].reshape(1, -1),
        p["angles_out"]["W"], p["angles_out"]["b"].reshape(1, -1),
        p["angle_scale"]["W"], p["angle_scale"]["b"].reshape(1, -1),
        p["sigs0"]["W"], p["sigs0"]["b"].reshape(1, -1),
        p["sigs1"]["W"], p["sigs1"]["b"].reshape(1, -1),
        p["sigs_out"]["W"], p["sigs_out"]["b"].reshape(1, -1),
    )
